# baseline (speedup 1.0000x reference)
import jax
import jax.numpy as jnp
from jax.experimental import pallas as pl


_BN = 4096


def _ydot(w, xt):
    return jax.lax.dot_general(
        w, xt, dimension_numbers=(((1,), (0,)), ((), ())),
        preferred_element_type=jnp.float32)


def _body(xt_ref, w_ref, b_ref, o_ref):
    i = pl.program_id(0)
    N = b_ref.shape[1]
    nb = N // _BN
    sl = pl.ds(i * _BN, _BN)

    @pl.when(i < nb)
    def _():
        yt = _ydot(w_ref[...], xt_ref[...])
        o_ref[:, sl] = jnp.transpose(yt, (1, 0)) + b_ref[:, sl]

    if N % _BN:
        tail = N - nb * _BN
        tsl = pl.ds(nb * _BN, tail)

        @pl.when(i == nb)
        def _():
            yt = _ydot(w_ref[pl.ds(0, tail), :], xt_ref[...])
            o_ref[:, tsl] = jnp.transpose(yt, (1, 0)) + b_ref[:, tsl]


def kernel(x, tgt, table_w0, table_b0):
    del tgt
    B, I, H = x.shape
    N = table_w0.shape[0]
    xt = x.reshape(B * I, H).T
    b2 = table_b0.reshape(1, N)
    out = pl.pallas_call(
        _body,
        grid=(pl.cdiv(N, _BN),),
        in_specs=[
            pl.BlockSpec((H, B * I), lambda i: (0, 0)),
            pl.BlockSpec((_BN, H), lambda i: (i, 0)),
            pl.BlockSpec((1, N), lambda i: (0, 0)),
        ],
        out_specs=pl.BlockSpec((B * I, N), lambda i: (0, 0)),
        out_shape=jax.ShapeDtypeStruct((B * I, N), jnp.float32),
    )(xt, table_w0, b2)
    return out.reshape(B, I, N)
